# baseline (speedup 1.0000x reference)
import functools

import jax
import jax.numpy as jnp
from jax import lax
from jax.experimental import pallas as pl
from jax.experimental.pallas import tpu as pltpu
from jax.experimental.pallas import tpu_sc as plsc
from jax.scipy.linalg import block_diag

_NC = 2
_NS = 16
_NW = _NC * _NS
_C = 40
_ROW = 1024
_HR = 64

_NTC = 4880
_NSC = 5120
_JCH = _NSC // (_NW * _C)


def _sc_agg(g2d, epsvec):
    mesh = plsc.VectorSubcoreMesh(core_axis_name="c", subcore_axis_name="s")

    @functools.partial(
        pl.kernel,
        out_type=jax.ShapeDtypeStruct((_NSC, _HR), jnp.float32),
        mesh=mesh,
        scratch_types=[
            pltpu.VMEM((_C, _ROW), jnp.float32),
            pltpu.VMEM((_C, _ROW), jnp.float32),
            pltpu.VMEM((_C, _HR), jnp.float32),
            pltpu.VMEM((16,), jnp.float32),
            pltpu.SemaphoreType.DMA,
            pltpu.SemaphoreType.DMA,
        ],
    )
    def agg(g_hbm, eps_hbm, h_hbm, buf0, buf1, obuf, epsv, sem0, sem1):
        wid = lax.axis_index("s") * _NC + lax.axis_index("c")
        pltpu.sync_copy(eps_hbm, epsv)
        ev = epsv[...]
        bufs = (buf0, buf1)
        sems = (sem0, sem1)

        def src(j):
            return g_hbm.at[pl.ds(_NTC + (wid + j * _NW) * _C, _C)]

        handles = [pltpu.async_copy(src(0), bufs[0], sems[0])]
        for j in range(_JCH):
            handles[j].wait()
            if j + 1 < _JCH:
                handles.append(
                    pltpu.async_copy(src(j + 1), bufs[(j + 1) % 2], sems[(j + 1) % 2]))
            b = bufs[j % 2]

            def cbody(c, carry):
                for k in range(4):
                    acc = b[c, pl.ds(k * 16, 16)] * ev
                    for s in range(1, 16):
                        acc = acc + b[c, pl.ds(s * 64 + k * 16, 16)]
                    obuf[c, pl.ds(k * 16, 16)] = acc
                return carry

            lax.fori_loop(0, _C, cbody, 0)
            pltpu.sync_copy(obuf, h_hbm.at[pl.ds((wid + j * _NW) * _C, _C)])

    return agg(g2d, epsvec)


def _fused_body(g_ref, A_ref, W1_ref, b1_ref, W2_ref, b2_ref,
                rW1_ref, rb1_ref, rW2_ref, rb2_ref, o_ref):
    gb = g_ref[...]
    hf = jnp.dot(gb, A_ref[...], preferred_element_type=jnp.float32)
    af = jnp.dot(hf, W1_ref[...], preferred_element_type=jnp.float32)
    b1v = b1_ref[...]
    u = jnp.maximum(af + b1v, 0.0) + jnp.maximum(b1v - af, 0.0)
    zf = jnp.dot(u, W2_ref[...], preferred_element_type=jnp.float32) + b2_ref[...]
    t = jnp.maximum(
        jnp.dot(zf, rW1_ref[...], preferred_element_type=jnp.float32) + rb1_ref[...],
        0.0)
    o_ref[...] = jnp.dot(t, rW2_ref[...], preferred_element_type=jnp.float32) + rb2_ref[...]


def _mlp_body(h_ref, W1_ref, b1_ref, W2_ref, b2_ref,
              rW1_ref, rb1_ref, rW2_ref, rb2_ref, o_ref):
    hf = h_ref[...]
    af = jnp.dot(hf, W1_ref[...], preferred_element_type=jnp.float32)
    b1v = b1_ref[...]
    u = jnp.maximum(af + b1v, 0.0) + jnp.maximum(b1v - af, 0.0)
    zf = jnp.dot(u, W2_ref[...], preferred_element_type=jnp.float32) + b2_ref[...]
    t = jnp.maximum(
        jnp.dot(zf, rW1_ref[...], preferred_element_type=jnp.float32) + rb1_ref[...],
        0.0)
    o_ref[...] = jnp.dot(t, rW2_ref[...], preferred_element_type=jnp.float32) + rb2_ref[...]


_WSPECS = [
    pl.BlockSpec((64, 256), lambda i: (0, 0)),
    pl.BlockSpec((1, 256), lambda i: (0, 0)),
    pl.BlockSpec((256, 128), lambda i: (0, 0)),
    pl.BlockSpec((1, 128), lambda i: (0, 0)),
    pl.BlockSpec((128, 64), lambda i: (0, 0)),
    pl.BlockSpec((1, 64), lambda i: (0, 0)),
    pl.BlockSpec((64, 32), lambda i: (0, 0)),
    pl.BlockSpec((1, 32), lambda i: (0, 0)),
]


def kernel(g, x, eps, enc_W1, enc_b1, enc_W2, enc_b2,
           rho_W1, rho_b1, rho_W2, rho_b2):
    B, N, S, M, D = g.shape
    H = enc_W1.shape[1]
    O = enc_W2.shape[1]
    MD = M * D
    NB = B * N

    g2d = g.reshape(NB, S * MD)
    epsvec = jnp.full((16,), 2.0 + eps, jnp.float32)

    h_sc = _sc_agg(g2d, epsvec)

    coef = jnp.ones((S,), g.dtype).at[0].add(1.0 + eps)
    A = (coef[:, None, None] * jnp.eye(MD, dtype=g.dtype)).reshape(S * MD, MD)
    W1big = block_diag(*([enc_W1] * M))
    b1big = jnp.tile(enc_b1, M)[None, :]
    W2big = block_diag(*([enc_W2] * M))
    b2big = jnp.tile(2.0 * enc_b2, M)[None, :]
    rb1 = rho_b1[None, :]
    rb2 = rho_b2[None, :]
    weights = (W1big, b1big, W2big, b2big, rho_W1, rb1, rho_W2, rb2)

    BN_A = 976
    out_tc = pl.pallas_call(
        _fused_body,
        grid=(_NTC // BN_A,),
        in_specs=[
            pl.BlockSpec((BN_A, S * MD), lambda i: (i, 0)),
            pl.BlockSpec((S * MD, MD), lambda i: (0, 0)),
        ] + _WSPECS,
        out_specs=pl.BlockSpec((BN_A, O), lambda i: (i, 0)),
        out_shape=jax.ShapeDtypeStruct((_NTC, O), g.dtype),
    )(g2d[:_NTC], A, *weights)

    BN_B = 1024
    out_sc = pl.pallas_call(
        _mlp_body,
        grid=(_NSC // BN_B,),
        in_specs=[pl.BlockSpec((BN_B, MD), lambda i: (i, 0))] + _WSPECS,
        out_specs=pl.BlockSpec((BN_B, O), lambda i: (i, 0)),
        out_shape=jax.ShapeDtypeStruct((_NSC, O), g.dtype),
    )(h_sc, *weights)

    out = jnp.concatenate([out_tc, out_sc], axis=0)
    return out.reshape(B, N, O)


# speedup vs baseline: 1.4204x; 1.4204x over previous
import functools

import jax
import jax.numpy as jnp
from jax import lax
from jax.experimental import pallas as pl
from jax.experimental.pallas import tpu as pltpu
from jax.experimental.pallas import tpu_sc as plsc
from jax.scipy.linalg import block_diag

_NC = 2
_NS = 16
_NW = _NC * _NS
_C = 40
_ROW = 1024
_HR = 64

_NTC = 4880
_NSC = 5120
_JCH = _NSC // (_NW * _C)


def _sc_agg(g2d, epsvec):
    mesh = plsc.VectorSubcoreMesh(core_axis_name="c", subcore_axis_name="s")

    @functools.partial(
        pl.kernel,
        out_type=jax.ShapeDtypeStruct((_NSC, _HR), jnp.float32),
        mesh=mesh,
        scratch_types=[
            pltpu.VMEM((_C, _ROW), jnp.float32),
            pltpu.VMEM((_C, _ROW), jnp.float32),
            pltpu.VMEM((_C, _HR), jnp.float32),
            pltpu.VMEM((16,), jnp.float32),
            pltpu.SemaphoreType.DMA,
            pltpu.SemaphoreType.DMA,
        ],
    )
    def agg(g_hbm, eps_hbm, h_hbm, buf0, buf1, obuf, epsv, sem0, sem1):
        wid = lax.axis_index("s") * _NC + lax.axis_index("c")
        pltpu.sync_copy(eps_hbm, epsv)
        ev = epsv[...]
        bufs = (buf0, buf1)
        sems = (sem0, sem1)

        def src(j):
            return g_hbm.at[pl.ds(_NTC + (wid + j * _NW) * _C, _C)]

        handles = [pltpu.async_copy(src(0), bufs[0], sems[0])]
        for j in range(_JCH):
            handles[j].wait()
            if j + 1 < _JCH:
                handles.append(
                    pltpu.async_copy(src(j + 1), bufs[(j + 1) % 2], sems[(j + 1) % 2]))
            b = bufs[j % 2]

            def cbody(c, carry):
                for k in range(4):
                    acc = b[c, pl.ds(k * 16, 16)] * ev
                    for s in range(1, 16):
                        acc = acc + b[c, pl.ds(s * 64 + k * 16, 16)]
                    obuf[c, pl.ds(k * 16, 16)] = acc
                return carry

            lax.fori_loop(0, _C, cbody, 0)
            pltpu.sync_copy(obuf, h_hbm.at[pl.ds((wid + j * _NW) * _C, _C)])

    return agg(g2d, epsvec)


def _fused_body(g_ref, A_ref, W1_ref, b1_ref, W2_ref, b2_ref,
                rW1_ref, rb1_ref, rW2_ref, rb2_ref, o_ref):
    gb = g_ref[...]
    hf = jnp.dot(gb, A_ref[...], preferred_element_type=jnp.float32)
    af = jnp.dot(hf, W1_ref[...], preferred_element_type=jnp.float32)
    b1v = b1_ref[...]
    u = jnp.maximum(af + b1v, 0.0) + jnp.maximum(b1v - af, 0.0)
    zf = jnp.dot(u, W2_ref[...], preferred_element_type=jnp.float32) + b2_ref[...]
    t = jnp.maximum(
        jnp.dot(zf, rW1_ref[...], preferred_element_type=jnp.float32) + rb1_ref[...],
        0.0)
    o_ref[...] = jnp.dot(t, rW2_ref[...], preferred_element_type=jnp.float32) + rb2_ref[...]


def _mlp_body(h_ref, W1_ref, b1_ref, W2_ref, b2_ref,
              rW1_ref, rb1_ref, rW2_ref, rb2_ref, o_ref):
    hf = h_ref[...]
    af = jnp.dot(hf, W1_ref[...], preferred_element_type=jnp.float32)
    b1v = b1_ref[...]
    u = jnp.maximum(af + b1v, 0.0) + jnp.maximum(b1v - af, 0.0)
    zf = jnp.dot(u, W2_ref[...], preferred_element_type=jnp.float32) + b2_ref[...]
    t = jnp.maximum(
        jnp.dot(zf, rW1_ref[...], preferred_element_type=jnp.float32) + rb1_ref[...],
        0.0)
    o_ref[...] = jnp.dot(t, rW2_ref[...], preferred_element_type=jnp.float32) + rb2_ref[...]


_WSPECS = [
    pl.BlockSpec((64, 256), lambda i: (0, 0)),
    pl.BlockSpec((1, 256), lambda i: (0, 0)),
    pl.BlockSpec((256, 128), lambda i: (0, 0)),
    pl.BlockSpec((1, 128), lambda i: (0, 0)),
    pl.BlockSpec((128, 64), lambda i: (0, 0)),
    pl.BlockSpec((1, 64), lambda i: (0, 0)),
    pl.BlockSpec((64, 32), lambda i: (0, 0)),
    pl.BlockSpec((1, 32), lambda i: (0, 0)),
]


def kernel(g, x, eps, enc_W1, enc_b1, enc_W2, enc_b2,
           rho_W1, rho_b1, rho_W2, rho_b2):
    B, N, S, M, D = g.shape
    H = enc_W1.shape[1]
    O = enc_W2.shape[1]
    MD = M * D
    NB = B * N

    g2d = g.reshape(NB, S * MD)
    epsvec = jnp.full((16,), 2.0 + eps, jnp.float32)

    h_sc = _sc_agg(g2d, epsvec)

    coef = jnp.ones((S,), g.dtype).at[0].add(1.0 + eps)
    A = (coef[:, None, None] * jnp.eye(MD, dtype=g.dtype)).reshape(S * MD, MD)
    W1big = block_diag(*([enc_W1] * M))
    b1big = jnp.tile(enc_b1, M)[None, :]
    W2big = block_diag(*([enc_W2] * M))
    b2big = jnp.tile(2.0 * enc_b2, M)[None, :]
    rb1 = rho_b1[None, :]
    rb2 = rho_b2[None, :]
    weights = (W1big, b1big, W2big, b2big, rho_W1, rb1, rho_W2, rb2)

    BN_A = 976
    out_tc = pl.pallas_call(
        _fused_body,
        grid=(_NTC // BN_A,),
        in_specs=[
            pl.BlockSpec((BN_A, S * MD), lambda i: (i, 0)),
            pl.BlockSpec((S * MD, MD), lambda i: (0, 0)),
        ] + _WSPECS,
        out_specs=pl.BlockSpec((BN_A, O), lambda i: (i, 0)),
        out_shape=jax.ShapeDtypeStruct((_NTC, O), g.dtype),
    )(g2d, A, *weights)

    BN_B = 1024
    out_sc = pl.pallas_call(
        _mlp_body,
        grid=(_NSC // BN_B,),
        in_specs=[pl.BlockSpec((BN_B, MD), lambda i: (i, 0))] + _WSPECS,
        out_specs=pl.BlockSpec((BN_B, O), lambda i: (i, 0)),
        out_shape=jax.ShapeDtypeStruct((_NSC, O), g.dtype),
    )(h_sc, *weights)

    out = jnp.concatenate([out_tc, out_sc], axis=0)
    return out.reshape(B, N, O)
